# baseline (speedup 1.0000x reference)
import functools
import math

import jax
import jax.numpy as jnp
from jax.experimental import pallas as pl
from jax.experimental.pallas import tpu as pltpu

SEG = 512
D = SEG * 3
E = 8
TOP_K = 2
H_EXP = SEG * 2
N_LAYERS = 3
N_HEADS = 4
OUT_DIM = 1000


def _bcast8(b):
    return jnp.broadcast_to(b[None, :], (8, b.shape[0]))


def _mm_body(x_ref, w_ref, b_ref, o_ref, acc_ref, *, act):
    k = pl.program_id(2)

    @pl.when(k == 0)
    def _():
        acc_ref[...] = jnp.zeros_like(acc_ref)

    acc_ref[...] += jnp.dot(x_ref[...].astype(jnp.bfloat16),
                            w_ref[...].astype(jnp.bfloat16),
                            preferred_element_type=jnp.float32)

    @pl.when(k == pl.num_programs(2) - 1)
    def _():
        r = acc_ref[...] + b_ref[0:1, :]
        if act == "relu":
            r = jnp.maximum(r, 0.0)
        elif act == "gelu":
            r = jax.nn.gelu(r)
        o_ref[...] = r


def _matmul(x, w, b, act=None):
    M, K = x.shape
    _, N = w.shape
    bm = min(M, 512)
    bn = 512 if N % 512 == 0 else 128
    bk = 512 if K % 512 == 0 else K
    grid = (M // bm, N // bn, K // bk)
    return pl.pallas_call(
        functools.partial(_mm_body, act=act),
        grid=grid,
        in_specs=[
            pl.BlockSpec((bm, bk), lambda m, n, k: (m, k)),
            pl.BlockSpec((bk, bn), lambda m, n, k: (k, n)),
            pl.BlockSpec((8, bn), lambda m, n, k: (0, n)),
        ],
        out_specs=pl.BlockSpec((bm, bn), lambda m, n, k: (m, n)),
        out_shape=jax.ShapeDtypeStruct((M, N), jnp.float32),
        scratch_shapes=[pltpu.VMEM((bm, bn), jnp.float32)],
        compiler_params=pltpu.CompilerParams(
            dimension_semantics=("parallel", "parallel", "arbitrary")),
    )(x, w, _bcast8(b))


def _amean_body(a_ref, o_ref):
    o_ref[...] = jnp.mean(a_ref[...], axis=2)


def _audio_mean(audio):
    B, C, T = audio.shape
    bm = 32
    return pl.pallas_call(
        _amean_body,
        grid=(B // bm,),
        in_specs=[pl.BlockSpec((bm, C, T), lambda m: (m, 0, 0))],
        out_specs=pl.BlockSpec((bm, C), lambda m: (m, 0)),
        out_shape=jax.ShapeDtypeStruct((B, C), jnp.float32),
    )(audio)


def _moe1_body(x_ref, w1_ref, b1_ref, h_ref, acc_ref):
    k = pl.program_id(3)

    @pl.when(k == 0)
    def _():
        acc_ref[...] = jnp.zeros_like(acc_ref)

    acc_ref[...] += jnp.dot(x_ref[...].astype(jnp.bfloat16),
                            w1_ref[0].astype(jnp.bfloat16),
                            preferred_element_type=jnp.float32)

    @pl.when(k == pl.num_programs(3) - 1)
    def _():
        h_ref[0] = jnp.maximum(acc_ref[...] + b1_ref[0, 0:1, :], 0.0)


def _moe2_body(h_ref, w2_ref, gp_ref, b2_ref, o_ref, acc_ref):
    e = pl.program_id(2)
    k = pl.program_id(3)

    @pl.when(jnp.logical_and(e == 0, k == 0))
    def _():
        acc_ref[...] = jnp.zeros_like(acc_ref)

    partial = jnp.dot(h_ref[0].astype(jnp.bfloat16),
                      w2_ref[0].astype(jnp.bfloat16),
                      preferred_element_type=jnp.float32)
    lane = jax.lax.broadcasted_iota(jnp.int32, gp_ref.shape, 1)
    g = jnp.sum(jnp.where(lane == e, gp_ref[...], 0.0), axis=1, keepdims=True)
    acc_ref[...] += g * partial

    @pl.when(jnp.logical_and(e == pl.num_programs(2) - 1,
                             k == pl.num_programs(3) - 1))
    def _():
        o_ref[...] = acc_ref[...] + jnp.dot(
            gp_ref[...], b2_ref[...], preferred_element_type=jnp.float32,
            precision=jax.lax.Precision.HIGHEST)


def _moe_dense(x, w1, b1, w2, b2, gates_pad):
    B = x.shape[0]
    bm, bn, bk = 512, 512, 512
    h = pl.pallas_call(
        _moe1_body,
        grid=(E, B // bm, H_EXP // bn, D // bk),
        in_specs=[
            pl.BlockSpec((bm, bk), lambda e, m, n, k: (m, k)),
            pl.BlockSpec((1, bk, bn), lambda e, m, n, k: (e, k, n)),
            pl.BlockSpec((1, 8, bn), lambda e, m, n, k: (e, 0, n)),
        ],
        out_specs=pl.BlockSpec((1, bm, bn), lambda e, m, n, k: (e, m, n)),
        out_shape=jax.ShapeDtypeStruct((E, B, H_EXP), jnp.float32),
        scratch_shapes=[pltpu.VMEM((bm, bn), jnp.float32)],
        compiler_params=pltpu.CompilerParams(
            dimension_semantics=("parallel", "parallel", "parallel", "arbitrary")),
    )(x, w1, jnp.broadcast_to(b1[:, None, :], (E, 8, H_EXP)))

    b2pad = jnp.zeros((128, D), jnp.float32).at[:E].set(b2)
    out = pl.pallas_call(
        _moe2_body,
        grid=(B // bm, D // bn, E, H_EXP // bk),
        in_specs=[
            pl.BlockSpec((1, bm, bk), lambda m, n, e, k: (e, m, k)),
            pl.BlockSpec((1, bk, bn), lambda m, n, e, k: (e, k, n)),
            pl.BlockSpec((bm, 128), lambda m, n, e, k: (m, 0)),
            pl.BlockSpec((128, bn), lambda m, n, e, k: (0, n)),
        ],
        out_specs=pl.BlockSpec((bm, bn), lambda m, n, e, k: (m, n)),
        out_shape=jax.ShapeDtypeStruct((B, D), jnp.float32),
        scratch_shapes=[pltpu.VMEM((bm, bn), jnp.float32)],
        compiler_params=pltpu.CompilerParams(
            dimension_semantics=("parallel", "parallel", "arbitrary", "arbitrary")),
    )(h, w2, gates_pad, b2pad)
    return out


def _ln_body(x_ref, r_ref, g_ref, b_ref, o_ref):
    y = x_ref[...] + r_ref[...]
    m = jnp.mean(y, axis=1, keepdims=True)
    v = jnp.mean((y - m) ** 2, axis=1, keepdims=True)
    o_ref[...] = (y - m) * jax.lax.rsqrt(v + 1e-5) * g_ref[0:1, :] + b_ref[0:1, :]


def _ln_residual(x, res, g, b):
    B, Dd = x.shape
    bm = 256
    return pl.pallas_call(
        _ln_body,
        grid=(B // bm,),
        in_specs=[
            pl.BlockSpec((bm, Dd), lambda m: (m, 0)),
            pl.BlockSpec((bm, Dd), lambda m: (m, 0)),
            pl.BlockSpec((8, Dd), lambda m: (0, 0)),
            pl.BlockSpec((8, Dd), lambda m: (0, 0)),
        ],
        out_specs=pl.BlockSpec((bm, Dd), lambda m: (m, 0)),
        out_shape=jax.ShapeDtypeStruct((B, Dd), jnp.float32),
    )(x, res, _bcast8(g), _bcast8(b))


def _fusion_body(xv_ref, xt_ref, xa_ref, wq_ref, wk_ref, wv_ref, wo_ref,
                 bq_ref, bk_ref, bv_ref, bo_ref, mh_ref, mht_ref, o_ref):
    hi = jax.lax.Precision.HIGHEST
    rb = lambda x: x.astype(jnp.bfloat16).astype(jnp.float32)
    dotbf = lambda a, b: jnp.dot(a.astype(jnp.bfloat16), b.astype(jnp.bfloat16),
                                 preferred_element_type=jnp.float32)
    xs = (xv_ref[...], xt_ref[...], xa_ref[...])
    q = [dotbf(x, wq_ref[...]) + bq_ref[0:1, :] for x in xs]
    k = [dotbf(x, wk_ref[...]) + bk_ref[0:1, :] for x in xs]
    v = [dotbf(x, wv_ref[...]) + bv_ref[0:1, :] for x in xs]
    mh = mh_ref[...]
    mht = mht_ref[...]
    scale = 1.0 / math.sqrt(SEG // N_HEADS)
    s = [[jnp.dot(rb(q[i]) * rb(k[j]), mh, preferred_element_type=jnp.float32,
                  precision=hi) * scale
          for j in range(3)] for i in range(3)]
    acc = None
    for i in range(3):
        mx = jnp.maximum(jnp.maximum(s[i][0], s[i][1]), s[i][2])
        ex = [jnp.exp(s[i][j] - mx) for j in range(3)]
        den = ex[0] + ex[1] + ex[2]
        for j in range(3):
            pb = jnp.dot(ex[j] / den, mht, preferred_element_type=jnp.float32,
                         precision=hi)
            term = rb(pb) * rb(v[j])
            acc = term if acc is None else acc + term
    acc = acc / 3.0
    o_ref[...] = dotbf(acc, wo_ref[...]) + bo_ref[0:1, :]


def _fusion(x, p):
    B = x.shape[0]
    bm = 256
    xv, xt, xa = x[:, :SEG], x[:, SEG:2 * SEG], x[:, 2 * SEG:]
    dh = SEG // N_HEADS
    head_of_dim = jnp.arange(SEG) // dh
    mh = (head_of_dim[:, None] == jnp.arange(128)[None, :]).astype(jnp.float32)
    mht = mh.T
    full = lambda: pl.BlockSpec((SEG, SEG), lambda m: (0, 0))
    bias = lambda: pl.BlockSpec((8, SEG), lambda m: (0, 0))
    return pl.pallas_call(
        _fusion_body,
        grid=(B // bm,),
        in_specs=[
            pl.BlockSpec((bm, SEG), lambda m: (m, 0)),
            pl.BlockSpec((bm, SEG), lambda m: (m, 0)),
            pl.BlockSpec((bm, SEG), lambda m: (m, 0)),
            full(), full(), full(), full(),
            bias(), bias(), bias(), bias(),
            pl.BlockSpec((SEG, 128), lambda m: (0, 0)),
            pl.BlockSpec((128, SEG), lambda m: (0, 0)),
        ],
        out_specs=pl.BlockSpec((bm, SEG), lambda m: (m, 0)),
        out_shape=jax.ShapeDtypeStruct((B, SEG), jnp.float32),
    )(xv, xt, xa, p["fus_wq"], p["fus_wk"], p["fus_wv"], p["fus_wo"],
      _bcast8(p["fus_bq"]), _bcast8(p["fus_bk"]), _bcast8(p["fus_bv"]),
      _bcast8(p["fus_bo"]), mh, mht)


def kernel(visual, text, audio, params):
    p = params
    B = visual.shape[0]

    v = visual.reshape(B, -1)
    v = _matmul(v, p["v_w1"], p["v_b1"], act="relu")
    v = _matmul(v, p["v_w2"], p["v_b2"], act="relu")
    v = _matmul(v, p["v_w3"], p["v_b3"])

    t = jnp.take(p["t_emb"], text, axis=0).mean(axis=1)
    t = _matmul(t, p["t_w1"], p["t_b1"], act="relu")
    t = _matmul(t, p["t_w2"], p["t_b2"])

    a = _audio_mean(audio)
    a = _matmul(a, p["a_w1"], p["a_b1"], act="relu")
    a = _matmul(a, p["a_w2"], p["a_b2"], act="relu")
    a = _matmul(a, p["a_w3"], p["a_b3"])

    x = jnp.concatenate([v, t, a], axis=-1)

    for i in range(N_LAYERS):
        rw = jnp.zeros((D, 128), jnp.float32).at[:, :E].set(p[f"moe{i}_router_w"])
        rb = jnp.zeros((128,), jnp.float32).at[:E].set(p[f"moe{i}_router_b"])
        logits = _matmul(x, rw, rb)[:, :E]
        probs = jax.nn.softmax(logits, axis=-1)
        topv, topi = jax.lax.top_k(probs, TOP_K)
        topv = topv / topv.sum(-1, keepdims=True)
        gates = jnp.zeros_like(probs).at[
            jnp.arange(B)[:, None], topi].set(topv)
        gates_pad = jnp.zeros((B, 128), jnp.float32).at[:, :E].set(gates)
        out = _moe_dense(x, p[f"moe{i}_w1"], p[f"moe{i}_b1"],
                         p[f"moe{i}_w2"], p[f"moe{i}_b2"], gates_pad)
        x = _ln_residual(x, out, p[f"ln{i}_g"], p[f"ln{i}_b"])

    f = _fusion(x, p)

    h = _matmul(f, p["h_w1"], p["h_b1"], act="gelu")
    w2p = jnp.zeros((SEG, 1024), jnp.float32).at[:, :OUT_DIM].set(p["h_w2"])
    b2p = jnp.zeros((1024,), jnp.float32).at[:OUT_DIM].set(p["h_b2"])
    return _matmul(h, w2p, b2p)[:, :OUT_DIM]
